# baseline (speedup 1.0000x reference)
import functools

import jax
import jax.numpy as jnp
from jax import lax
from jax.experimental import pallas as pl
from jax.experimental.pallas import tpu as pltpu
from jax.experimental.pallas import tpu_sc as plsc

NC = 2
NS = 16
LANES = 16
NW = NC * NS

N_NODES = 10000
N_EDGES = 320000
N_CAND = 262144
D = 128

PAD_NODES = 10240
ROWS_PER_TILE = PAD_NODES // NS

CHUNK = 128
EDGES_PAD = 327680
EDGES_PER_W = EDGES_PAD // NW
NCHUNK = EDGES_PER_W // CHUNK

CNT_W = 128

DCHUNK = 128
CAND_PER_W = N_CAND // NW
DNCHUNK = CAND_PER_W // DCHUNK

_MESH = plsc.VectorSubcoreMesh(
    core_axis_name="c", subcore_axis_name="s", num_cores=NC, num_subcores=NS
)


@functools.partial(
    pl.kernel,
    out_type=jax.ShapeDtypeStruct((NC, PAD_NODES, D), jnp.float32),
    mesh=_MESH,
    name="sc_agg",
    scratch_types=[
        pltpu.VMEM((NCHUNK, CHUNK), jnp.int32),
        pltpu.VMEM((NCHUNK, CHUNK), jnp.int32),
        pltpu.VMEM((CHUNK, D), jnp.float32),
        pltpu.VMEM_SHARED((PAD_NODES, D), jnp.float32),
        pltpu.SemaphoreType.DMA,
    ],
)
def _agg(x_hbm, srcs_hbm, dsts_hbm, zrow_hbm,
         acc_out, idx_s, idx_d, rows_a, acc_sh, sem_a):
    cid = lax.axis_index("c")
    sid = lax.axis_index("s")
    wid = cid * NS + sid
    r0 = sid * ROWS_PER_TILE

    pltpu.sync_copy(zrow_hbm, acc_sh.at[pl.ds(r0, ROWS_PER_TILE)])
    pltpu.sync_copy(srcs_hbm.at[pl.ds(wid * NCHUNK, NCHUNK)], idx_s)
    pltpu.sync_copy(dsts_hbm.at[pl.ds(wid * NCHUNK, NCHUNK)], idx_d)
    plsc.subcore_barrier()

    def chunk(k, carry):
        pltpu.async_copy(x_hbm.at[idx_s.at[k]], rows_a, sem_a).wait()
        pltpu.sync_copy(rows_a, acc_sh.at[idx_d.at[k]], add=True)
        return carry

    lax.fori_loop(0, NCHUNK, chunk, 0)
    plsc.subcore_barrier()

    pltpu.sync_copy(acc_sh.at[pl.ds(r0, ROWS_PER_TILE)],
                    acc_out.at[cid, pl.ds(r0, ROWS_PER_TILE)])


@functools.partial(
    pl.kernel,
    out_type=jax.ShapeDtypeStruct((NC, PAD_NODES, CNT_W), jnp.float32),
    mesh=_MESH,
    name="sc_degree",
    scratch_types=[
        pltpu.VMEM((NCHUNK, CHUNK), jnp.int32),
        pltpu.VMEM((CHUNK, CNT_W), jnp.float32),
        pltpu.VMEM_SHARED((PAD_NODES, CNT_W), jnp.float32),
    ],
)
def _degree(dsts_hbm, zcnt_hbm, ones_hbm,
            cnt_out, idx_d, ones_v, cnt_sh):
    cid = lax.axis_index("c")
    sid = lax.axis_index("s")
    wid = cid * NS + sid
    r0 = sid * ROWS_PER_TILE

    pltpu.sync_copy(zcnt_hbm, cnt_sh.at[pl.ds(r0, ROWS_PER_TILE)])
    pltpu.sync_copy(ones_hbm, ones_v)
    pltpu.sync_copy(dsts_hbm.at[pl.ds(wid * NCHUNK, NCHUNK)], idx_d)
    plsc.subcore_barrier()

    def chunk(k, carry):
        pltpu.sync_copy(ones_v, cnt_sh.at[idx_d.at[k]], add=True)
        return carry

    lax.fori_loop(0, NCHUNK, chunk, 0)
    plsc.subcore_barrier()

    pltpu.sync_copy(cnt_sh.at[pl.ds(r0, ROWS_PER_TILE)],
                    cnt_out.at[cid, pl.ds(r0, ROWS_PER_TILE)])


def _dense_body(relu, p0, p1, c0, c1, x, wl, bl, wr, o):
    agg = p0[...] + p1[...]
    cnt = c0[:, 0:1] + c1[:, 0:1]
    mean = agg / jnp.maximum(cnt, 1.0)
    h = (lax.dot_general(mean, wl[...], (((1,), (1,)), ((), ())),
                         preferred_element_type=jnp.float32)
         + bl[...]
         + lax.dot_general(x[...], wr[...], (((1,), (1,)), ((), ())),
                           preferred_element_type=jnp.float32))
    o[...] = jnp.maximum(h, 0.0) if relu else h


def _dense_layer(p0, p1, c0, c1, x, wl, bl, wr, relu):
    rb = 2048
    return pl.pallas_call(
        functools.partial(_dense_body, relu),
        grid=(PAD_NODES // rb,),
        in_specs=[
            pl.BlockSpec((rb, D), lambda i: (i, 0)),
            pl.BlockSpec((rb, D), lambda i: (i, 0)),
            pl.BlockSpec((rb, CNT_W), lambda i: (i, 0)),
            pl.BlockSpec((rb, CNT_W), lambda i: (i, 0)),
            pl.BlockSpec((rb, D), lambda i: (i, 0)),
            pl.BlockSpec((D, D), lambda i: (0, 0)),
            pl.BlockSpec((1, D), lambda i: (0, 0)),
            pl.BlockSpec((D, D), lambda i: (0, 0)),
        ],
        out_specs=pl.BlockSpec((rb, D), lambda i: (i, 0)),
        out_shape=jax.ShapeDtypeStruct((PAD_NODES, D), jnp.float32),
    )(p0, p1, c0, c1, x, wl, bl, wr)


@functools.partial(
    pl.kernel,
    out_type=jax.ShapeDtypeStruct((N_CAND, LANES), jnp.float32),
    mesh=_MESH,
    name="sc_decode",
    scratch_types=[
        pltpu.VMEM((DNCHUNK, DCHUNK), jnp.int32),
        pltpu.VMEM((DNCHUNK, DCHUNK), jnp.int32),
        pltpu.VMEM((DCHUNK, D), jnp.float32),
        pltpu.VMEM((DCHUNK, D), jnp.float32),
        pltpu.VMEM((DCHUNK, LANES), jnp.float32),
        pltpu.SemaphoreType.DMA,
        pltpu.SemaphoreType.DMA,
    ],
)
def _decode(z_hbm, s2d_hbm, d2d_hbm, out_hbm,
            idx_s, idx_d, srow, drow, red, sem1, sem2):
    cid = lax.axis_index("c")
    sid = lax.axis_index("s")
    wid = cid * NS + sid
    pltpu.sync_copy(s2d_hbm.at[pl.ds(wid * DNCHUNK, DNCHUNK)], idx_s)
    pltpu.sync_copy(d2d_hbm.at[pl.ds(wid * DNCHUNK, DNCHUNK)], idx_d)
    base_out = wid * CAND_PER_W

    def chunk(c, carry):
        cp1 = pltpu.async_copy(z_hbm.at[idx_s.at[c]], srow, sem1)
        cp2 = pltpu.async_copy(z_hbm.at[idx_d.at[c]], drow, sem2)
        cp1.wait()
        cp2.wait()

        def rbody(i, carry2):
            acc = srow[i, pl.ds(0, LANES)] * drow[i, pl.ds(0, LANES)]
            for jj in range(1, D // LANES):
                acc = acc + (srow[i, pl.ds(jj * LANES, LANES)]
                             * drow[i, pl.ds(jj * LANES, LANES)])
            red[i, :] = acc
            return carry2

        lax.fori_loop(0, DCHUNK, rbody, 0)
        off = pl.multiple_of(base_out + c * DCHUNK, DCHUNK)
        pltpu.sync_copy(red, out_hbm.at[pl.ds(off, DCHUNK)])
        return carry

    lax.fori_loop(0, DNCHUNK, chunk, 0)


def _reduce_body(r, o):
    o[...] = jnp.sum(r[...], axis=-1)


def _reduce_lanes(red3d):
    rb = 256
    n = red3d.shape[0]
    return pl.pallas_call(
        _reduce_body,
        grid=(n // rb,),
        in_specs=[pl.BlockSpec((rb, D, LANES), lambda i: (i, 0, 0))],
        out_specs=pl.BlockSpec((rb, D), lambda i: (i, 0)),
        out_shape=jax.ShapeDtypeStruct((n, D), jnp.float32),
    )(red3d)


def kernel(node_features, W1_l, b1_l, W1_r, W2_l, b2_l, W2_r,
           edge_index, edge_label_index):
    x = jnp.pad(node_features.astype(jnp.float32),
                ((0, PAD_NODES - N_NODES), (0, 0)))
    ei = edge_index.astype(jnp.int32)
    eli = edge_label_index.astype(jnp.int32)

    pad_e = EDGES_PAD - N_EDGES
    srcs = jnp.pad(ei[0], (0, pad_e)).reshape(NW * NCHUNK, CHUNK)
    junk = N_NODES + (jnp.arange(pad_e, dtype=jnp.int32)
                      % (PAD_NODES - N_NODES))
    dsts = jnp.concatenate([ei[1], junk]).reshape(NW * NCHUNK, CHUNK)

    zrow = jnp.zeros((ROWS_PER_TILE, D), jnp.float32)
    zcnt = jnp.zeros((ROWS_PER_TILE, CNT_W), jnp.float32)
    ones = jnp.ones((CHUNK, CNT_W), jnp.float32)

    cnt = _degree(dsts, zcnt, ones)
    agg1 = _agg(x, srcs, dsts, zrow)
    h = _dense_layer(agg1[0], agg1[1], cnt[0], cnt[1], x,
                     W1_l, b1_l.reshape(1, D), W1_r, relu=True)
    agg2 = _agg(h, srcs, dsts, zrow)
    z = _dense_layer(agg2[0], agg2[1], cnt[0], cnt[1], h,
                     W2_l, b2_l.reshape(1, D), W2_r, relu=False)

    s2d = eli[0].reshape(NW * DNCHUNK, DCHUNK)
    d2d = eli[1].reshape(NW * DNCHUNK, DCHUNK)
    red = _decode(z, s2d, d2d)
    logits2d = _reduce_lanes(red.reshape(N_CAND // D, D, LANES))
    return logits2d.reshape(N_CAND)


# speedup vs baseline: 1.1478x; 1.1478x over previous
import functools

import jax
import jax.numpy as jnp
from jax import lax
from jax.experimental import pallas as pl
from jax.experimental.pallas import tpu as pltpu
from jax.experimental.pallas import tpu_sc as plsc

NC = 2
NS = 16
LANES = 16
NW = NC * NS

N_NODES = 10000
N_EDGES = 320000
N_CAND = 262144
D = 128

PAD_NODES = 10240
ROWS_PER_TILE = PAD_NODES // NS

CHUNK = 128
EDGES_PAD = 327680
EDGES_PER_W = EDGES_PAD // NW
NCHUNK = EDGES_PER_W // CHUNK
ZTILE = 80

CNT_W = 128

DCHUNK = 128
CAND_PER_W = N_CAND // NW
DNCHUNK = CAND_PER_W // DCHUNK

_MESH = plsc.VectorSubcoreMesh(
    core_axis_name="c", subcore_axis_name="s", num_cores=NC, num_subcores=NS
)


@functools.partial(
    pl.kernel,
    out_type=jax.ShapeDtypeStruct((NC, PAD_NODES, D), jnp.float32),
    mesh=_MESH,
    name="sc_agg",
    scratch_types=[
        pltpu.VMEM((NCHUNK, CHUNK), jnp.int32),
        pltpu.VMEM((NCHUNK, CHUNK), jnp.int32),
        pltpu.VMEM((CHUNK, D), jnp.float32),
        pltpu.VMEM_SHARED((PAD_NODES, D), jnp.float32),
        pltpu.SemaphoreType.DMA,
        pltpu.SemaphoreType.DMA,
    ],
)
def _agg(x_hbm, srcs_hbm, dsts_hbm, acc_out,
         idx_s, idx_d, rows_a, acc_sh, gsem_a, ssem_a):
    cid = lax.axis_index("c")
    sid = lax.axis_index("s")
    wid = cid * NS + sid
    r0 = sid * ROWS_PER_TILE

    def zrow(i, carry):
        for j in range(D // LANES):
            rows_a[i, pl.ds(j * LANES, LANES)] = jnp.zeros((LANES,),
                                                           jnp.float32)
        return carry

    lax.fori_loop(0, ZTILE, zrow, 0)
    for j in range(ROWS_PER_TILE // ZTILE):
        pltpu.sync_copy(rows_a.at[pl.ds(0, ZTILE)],
                        acc_sh.at[pl.ds(r0 + j * ZTILE, ZTILE)])
    pltpu.sync_copy(srcs_hbm.at[pl.ds(wid * NCHUNK, NCHUNK)], idx_s)
    pltpu.sync_copy(dsts_hbm.at[pl.ds(wid * NCHUNK, NCHUNK)], idx_d)
    plsc.subcore_barrier()

    def chunk(k, carry):
        pltpu.async_copy(x_hbm.at[idx_s.at[k]], rows_a, gsem_a).wait()
        pltpu.sync_copy(rows_a, acc_sh.at[idx_d.at[k]], add=True)
        return carry

    lax.fori_loop(0, NCHUNK, chunk, 0)
    plsc.subcore_barrier()

    pltpu.sync_copy(acc_sh.at[pl.ds(r0, ROWS_PER_TILE)],
                    acc_out.at[cid, pl.ds(r0, ROWS_PER_TILE)])


@functools.partial(
    pl.kernel,
    out_type=jax.ShapeDtypeStruct((NC, PAD_NODES, CNT_W), jnp.float32),
    mesh=_MESH,
    name="sc_degree",
    scratch_types=[
        pltpu.VMEM((NCHUNK, CHUNK), jnp.int32),
        pltpu.VMEM((CHUNK, CNT_W), jnp.float32),
        pltpu.VMEM_SHARED((PAD_NODES, CNT_W), jnp.float32),
    ],
)
def _degree(dsts_hbm, zcnt_hbm, ones_hbm,
            cnt_out, idx_d, ones_v, cnt_sh):
    cid = lax.axis_index("c")
    sid = lax.axis_index("s")
    wid = cid * NS + sid
    r0 = sid * ROWS_PER_TILE

    pltpu.sync_copy(zcnt_hbm, cnt_sh.at[pl.ds(r0, ROWS_PER_TILE)])
    pltpu.sync_copy(ones_hbm, ones_v)
    pltpu.sync_copy(dsts_hbm.at[pl.ds(wid * NCHUNK, NCHUNK)], idx_d)
    plsc.subcore_barrier()

    def chunk(k, carry):
        pltpu.sync_copy(ones_v, cnt_sh.at[idx_d.at[k]], add=True)
        return carry

    lax.fori_loop(0, NCHUNK, chunk, 0)
    plsc.subcore_barrier()

    pltpu.sync_copy(cnt_sh.at[pl.ds(r0, ROWS_PER_TILE)],
                    cnt_out.at[cid, pl.ds(r0, ROWS_PER_TILE)])


def _dense_body(relu, p0, p1, c0, c1, x, wl, bl, wr, o):
    agg = p0[...] + p1[...]
    cnt = c0[:, 0:1] + c1[:, 0:1]
    mean = agg / jnp.maximum(cnt, 1.0)
    h = (lax.dot_general(mean, wl[...], (((1,), (1,)), ((), ())),
                         preferred_element_type=jnp.float32)
         + bl[...]
         + lax.dot_general(x[...], wr[...], (((1,), (1,)), ((), ())),
                           preferred_element_type=jnp.float32))
    o[...] = jnp.maximum(h, 0.0) if relu else h


def _dense_layer(p0, p1, c0, c1, x, wl, bl, wr, relu):
    rb = 2048
    return pl.pallas_call(
        functools.partial(_dense_body, relu),
        grid=(PAD_NODES // rb,),
        in_specs=[
            pl.BlockSpec((rb, D), lambda i: (i, 0)),
            pl.BlockSpec((rb, D), lambda i: (i, 0)),
            pl.BlockSpec((rb, CNT_W), lambda i: (i, 0)),
            pl.BlockSpec((rb, CNT_W), lambda i: (i, 0)),
            pl.BlockSpec((rb, D), lambda i: (i, 0)),
            pl.BlockSpec((D, D), lambda i: (0, 0)),
            pl.BlockSpec((1, D), lambda i: (0, 0)),
            pl.BlockSpec((D, D), lambda i: (0, 0)),
        ],
        out_specs=pl.BlockSpec((rb, D), lambda i: (i, 0)),
        out_shape=jax.ShapeDtypeStruct((PAD_NODES, D), jnp.float32),
    )(p0, p1, c0, c1, x, wl, bl, wr)


@functools.partial(
    pl.kernel,
    out_type=jax.ShapeDtypeStruct((N_CAND, LANES), jnp.float32),
    mesh=_MESH,
    name="sc_decode",
    scratch_types=[
        pltpu.VMEM((DNCHUNK, DCHUNK), jnp.int32),
        pltpu.VMEM((DNCHUNK, DCHUNK), jnp.int32),
        pltpu.VMEM((DCHUNK, D), jnp.float32),
        pltpu.VMEM((DCHUNK, D), jnp.float32),
        pltpu.VMEM((DCHUNK, D), jnp.float32),
        pltpu.VMEM((DCHUNK, D), jnp.float32),
        pltpu.VMEM((DCHUNK, LANES), jnp.float32),
        pltpu.SemaphoreType.DMA,
        pltpu.SemaphoreType.DMA,
        pltpu.SemaphoreType.DMA,
        pltpu.SemaphoreType.DMA,
        pltpu.SemaphoreType.DMA,
    ],
)
def _decode(z_hbm, s2d_hbm, d2d_hbm, out_hbm,
            idx_s, idx_d, srow_a, drow_a, srow_b, drow_b, red,
            sem1a, sem2a, sem1b, sem2b, osem):
    cid = lax.axis_index("c")
    sid = lax.axis_index("s")
    wid = cid * NS + sid
    pltpu.sync_copy(s2d_hbm.at[pl.ds(wid * DNCHUNK, DNCHUNK)], idx_s)
    pltpu.sync_copy(d2d_hbm.at[pl.ds(wid * DNCHUNK, DNCHUNK)], idx_d)
    base_out = wid * CAND_PER_W

    def dots(srow, drow, c):
        def rbody(i, carry2):
            acc = srow[i, pl.ds(0, LANES)] * drow[i, pl.ds(0, LANES)]
            for jj in range(1, D // LANES):
                acc = acc + (srow[i, pl.ds(jj * LANES, LANES)]
                             * drow[i, pl.ds(jj * LANES, LANES)])
            red[i, :] = acc
            return carry2

        lax.fori_loop(0, DCHUNK, rbody, 0)
        off = pl.multiple_of(base_out + c * DCHUNK, DCHUNK)
        pltpu.sync_copy(red, out_hbm.at[pl.ds(off, DCHUNK)])

    pltpu.async_copy(z_hbm.at[idx_s.at[0]], srow_a, sem1a)
    pltpu.async_copy(z_hbm.at[idx_d.at[0]], drow_a, sem2a)

    def pair(t, carry):
        c0 = t * 2
        pltpu.async_copy(z_hbm.at[idx_s.at[c0 + 1]], srow_b, sem1b)
        pltpu.async_copy(z_hbm.at[idx_d.at[c0 + 1]], drow_b, sem2b)
        pltpu.make_async_copy(z_hbm.at[idx_s.at[c0]], srow_a, sem1a).wait()
        pltpu.make_async_copy(z_hbm.at[idx_d.at[c0]], drow_a, sem2a).wait()
        dots(srow_a, drow_a, c0)

        @pl.when(t < DNCHUNK // 2 - 1)
        def _():
            pltpu.async_copy(z_hbm.at[idx_s.at[c0 + 2]], srow_a, sem1a)
            pltpu.async_copy(z_hbm.at[idx_d.at[c0 + 2]], drow_a, sem2a)

        pltpu.make_async_copy(z_hbm.at[idx_s.at[c0 + 1]], srow_b, sem1b).wait()
        pltpu.make_async_copy(z_hbm.at[idx_d.at[c0 + 1]], drow_b, sem2b).wait()
        dots(srow_b, drow_b, c0 + 1)
        return carry

    lax.fori_loop(0, DNCHUNK // 2, pair, 0)


def _reduce_body(r, o):
    o[...] = jnp.sum(r[...], axis=-1)


def _reduce_lanes(red3d):
    rb = 256
    n = red3d.shape[0]
    return pl.pallas_call(
        _reduce_body,
        grid=(n // rb,),
        in_specs=[pl.BlockSpec((rb, D, LANES), lambda i: (i, 0, 0))],
        out_specs=pl.BlockSpec((rb, D), lambda i: (i, 0)),
        out_shape=jax.ShapeDtypeStruct((n, D), jnp.float32),
    )(red3d)


def kernel(node_features, W1_l, b1_l, W1_r, W2_l, b2_l, W2_r,
           edge_index, edge_label_index):
    x = jnp.pad(node_features.astype(jnp.float32),
                ((0, PAD_NODES - N_NODES), (0, 0)))
    ei = edge_index.astype(jnp.int32)
    eli = edge_label_index.astype(jnp.int32)

    pad_e = EDGES_PAD - N_EDGES
    srcs = jnp.pad(ei[0], (0, pad_e)).reshape(NW * NCHUNK, CHUNK)
    dsts = jnp.pad(ei[1], (0, pad_e),
                   constant_values=PAD_NODES - 1).reshape(NW * NCHUNK, CHUNK)

    zcnt = jnp.zeros((ROWS_PER_TILE, CNT_W), jnp.float32)
    ones = jnp.ones((CHUNK, CNT_W), jnp.float32)

    cnt = _degree(dsts, zcnt, ones)
    agg1 = _agg(x, srcs, dsts)
    h = _dense_layer(agg1[0], agg1[1], cnt[0], cnt[1], x,
                     W1_l, b1_l.reshape(1, D), W1_r, relu=True)
    agg2 = _agg(h, srcs, dsts)
    z = _dense_layer(agg2[0], agg2[1], cnt[0], cnt[1], h,
                     W2_l, b2_l.reshape(1, D), W2_r, relu=False)

    s2d = eli[0].reshape(NW * DNCHUNK, DCHUNK)
    d2d = eli[1].reshape(NW * DNCHUNK, DCHUNK)
    red = _decode(z, s2d, d2d)
    logits2d = _reduce_lanes(red.reshape(N_CAND // D, D, LANES))
    return logits2d.reshape(N_CAND)
